# baseline (speedup 1.0000x reference)
import functools

import jax
import jax.numpy as jnp
from jax import lax
from jax.experimental import pallas as pl
from jax.experimental.pallas import tpu as pltpu
from jax.experimental.pallas import tpu_sc as plsc


def kernel(x_user, x_track_pos, x_track_neg, user_mat, track_mat):
    B = x_user.shape[0]
    D = user_mat.shape[1]
    info = plsc.get_sparse_core_info()
    NW = info.num_cores * info.num_subcores
    b_per_w = B // NW

    mesh = plsc.VectorSubcoreMesh(core_axis_name="c", subcore_axis_name="s")
    out_sds = jax.ShapeDtypeStruct((B, D), jnp.float32)

    @functools.partial(
        pl.kernel,
        mesh=mesh,
        out_type=(out_sds, out_sds, out_sds),
        scratch_types=[
            pltpu.VMEM((b_per_w,), jnp.int32),
            pltpu.VMEM((b_per_w, D), jnp.float32),
            pltpu.SemaphoreType.DMA,
        ],
        compiler_params=pltpu.CompilerParams(use_tc_tiling_on_sc=False),
    )
    def gather3(xu, xp, xn, um, tm, out_u, out_p, out_n, idx_v, rows_v, sem):
        wid = lax.axis_index("s") * info.num_cores + lax.axis_index("c")
        base = wid * b_per_w
        for idx_hbm, table, out in ((xu, um, out_u), (xp, tm, out_p), (xn, tm, out_n)):
            pltpu.sync_copy(idx_hbm.at[pl.ds(base, b_per_w)], idx_v)
            pltpu.async_copy(table.at[idx_v], rows_v, sem).wait()
            pltpu.sync_copy(rows_v, out.at[pl.ds(base, b_per_w)])

    return tuple(gather3(x_user, x_track_pos, x_track_neg, user_mat, track_mat))
